# Initial kernel scaffold; baseline (speedup 1.0000x reference)
import jax
import jax.numpy as jnp
from jax.experimental import pallas as pl


def kernel(points, params):
    raise NotImplementedError("write your pallas kernel here")


# speedup vs baseline: 3.9169x; 3.9169x over previous
import functools

import jax
import jax.numpy as jnp
from jax.experimental import pallas as pl

_B = 2
_N = 8192
_F32 = jnp.float32
_PREC = None


def _split3(a):
    a1 = a.astype(jnp.bfloat16).astype(_F32)
    r1 = a - a1
    a2 = r1.astype(jnp.bfloat16).astype(_F32)
    a3 = r1 - a2
    return a1, a2, a3


def _exact_gather(onehot, a):
    a1, a2, a3 = _split3(a)
    d = functools.partial(jnp.dot, preferred_element_type=_F32)
    return (d(onehot, a1) + d(onehot, a2)) + d(onehot, a3)


def _fps_body(x_ref, y_ref, z_ref, ox_ref, oy_ref, oz_ref, *, npoint, n):
    x = x_ref[0]
    y = y_ref[0]
    z = z_ref[0]
    rows = x.shape[0]
    lin = (jax.lax.broadcasted_iota(jnp.int32, (rows, 128), 0) * 128
           + jax.lax.broadcasted_iota(jnp.int32, (rows, 128), 1))
    orows = npoint // 128
    olin = (jax.lax.broadcasted_iota(jnp.int32, (orows, 128), 0) * 128
            + jax.lax.broadcasted_iota(jnp.int32, (orows, 128), 1))

    def body(i, state):
        dists, far, ox, oy, oz = state
        sel = lin == far
        cx = jnp.sum(jnp.where(sel, x, 0.0))
        cy = jnp.sum(jnp.where(sel, y, 0.0))
        cz = jnp.sum(jnp.where(sel, z, 0.0))
        dx = x - cx
        dy = y - cy
        dz = z - cz
        d = dx * dx + dy * dy + dz * dz
        dists = jnp.minimum(dists, d)
        step = olin == i
        ox = jnp.where(step, cx, ox)
        oy = jnp.where(step, cy, oy)
        oz = jnp.where(step, cz, oz)
        mx = jnp.max(dists)
        far = jnp.min(jnp.where(dists == mx, lin, n)).astype(jnp.int32)
        return dists, far, ox, oy, oz

    init = (jnp.full((rows, 128), 1e10, _F32), jnp.int32(0),
            jnp.zeros((orows, 128), _F32), jnp.zeros((orows, 128), _F32),
            jnp.zeros((orows, 128), _F32))
    _, _, ox, oy, oz = jax.lax.fori_loop(0, npoint, body, init)
    ox_ref[0] = ox
    oy_ref[0] = oy
    oz_ref[0] = oz


def _fps(xyz, npoint):
    b, n, _ = xyz.shape
    rows = n // 128
    orows = npoint // 128
    xs = xyz[:, :, 0].reshape(b, rows, 128)
    ys = xyz[:, :, 1].reshape(b, rows, 128)
    zs = xyz[:, :, 2].reshape(b, rows, 128)
    spec_in = pl.BlockSpec((1, rows, 128), lambda i: (i, 0, 0))
    spec_out = pl.BlockSpec((1, orows, 128), lambda i: (i, 0, 0))
    out_sd = jax.ShapeDtypeStruct((b, orows, 128), _F32)
    ox, oy, oz = pl.pallas_call(
        functools.partial(_fps_body, npoint=npoint, n=n),
        grid=(b,),
        in_specs=[spec_in, spec_in, spec_in],
        out_specs=(spec_out, spec_out, spec_out),
        out_shape=(out_sd, out_sd, out_sd),
    )(xs, ys, zs)
    return jnp.stack([ox.reshape(b, npoint), oy.reshape(b, npoint),
                      oz.reshape(b, npoint)], axis=-1)


def _sa_scale_body(xt_ref, p_ref, c_ref, *refs, radius, nsample, nlayers):
    out_ref = refs[-1]
    wrefs = refs[:-1]
    xt = xt_ref[0]
    pts = p_ref[0]
    c = c_ref[0]
    n = pts.shape[0]
    mb = c.shape[0]
    cin = pts.shape[1]

    px = xt[0:1, :]
    py = xt[1:2, :]
    pz = xt[2:3, :]
    cx = c[:, 0:1]
    cy = c[:, 1:2]
    cz = c[:, 2:3]
    ddx = cx - px
    ddy = cy - py
    ddz = cz - pz
    dist = ddx * ddx + ddy * ddy + ddz * ddz
    r2 = radius * radius
    cur = jnp.where(dist <= r2, dist, 1e10)

    iota_n = jax.lax.broadcasted_iota(jnp.int32, (mb, n), 1)
    cpad = jnp.concatenate([c, jnp.zeros((mb, cin - 3), _F32)], axis=1)

    p1, p2, p3 = _split3(pts)

    def extract(cur):
        m = jnp.min(cur, axis=1, keepdims=True)
        idx = jnp.min(jnp.where(cur == m, iota_n, n), axis=1,
                      keepdims=True)
        onehot = (iota_n == idx)
        oh = onehot.astype(_F32)
        d = functools.partial(jnp.dot, preferred_element_type=_F32)
        g = (d(oh, p1) + d(oh, p2)) + d(oh, p3)
        return m, onehot, g

    def mlp(g):
        h = g - cpad
        for li in range(nlayers):
            w = wrefs[2 * li][...]
            bb = wrefs[2 * li + 1][...]
            h = jnp.dot(h, w, preferred_element_type=_F32) + bb
            h = jnp.maximum(h, 0.0)
        return h

    m0, onehot0, g0 = extract(cur)
    cur = jnp.where(onehot0, 1e10, cur)
    pooled0 = mlp(g0)

    def body(_, carry):
        cur, pooled = carry
        m, onehot, g = extract(cur)
        cur = jnp.where(onehot, 1e10, cur)
        gsel = jnp.where(m <= r2, g, g0)
        h = mlp(gsel)
        return cur, jnp.maximum(pooled, h)

    _, pooled = jax.lax.fori_loop(1, nsample, body, (cur, pooled0))
    out_ref[0] = pooled


def _sa_scale(xyzt, pts, centers, layers, radius, nsample, mb):
    b, _, n = xyzt.shape
    m = centers.shape[1]
    cout = layers[-1]["W"].shape[1]
    nlayers = len(layers)
    args = [xyzt, pts, centers]
    in_specs = [
        pl.BlockSpec((1, 3, n), lambda i, j: (i, 0, 0)),
        pl.BlockSpec((1, n, pts.shape[2]), lambda i, j: (i, 0, 0)),
        pl.BlockSpec((1, mb, 3), lambda i, j: (i, j, 0)),
    ]
    for l in layers:
        args.append(l["W"])
        args.append(l["b"].reshape(1, -1))
        in_specs.append(pl.BlockSpec(l["W"].shape, lambda i, j: (0, 0)))
        in_specs.append(pl.BlockSpec((1, l["W"].shape[1]),
                                     lambda i, j: (0, 0)))
    out = pl.pallas_call(
        functools.partial(_sa_scale_body, radius=radius, nsample=nsample,
                          nlayers=nlayers),
        grid=(b, m // mb),
        in_specs=in_specs,
        out_specs=pl.BlockSpec((1, mb, cout), lambda i, j: (i, j, 0)),
        out_shape=jax.ShapeDtypeStruct((b, m, cout), _F32),
    )(*args)
    return out


def _mlp_body(*refs, nx, nlayers, last_linear):
    out_ref = refs[-1]
    if nx == 1:
        h = refs[0][...]
    else:
        h = jnp.concatenate([refs[i][...] for i in range(nx)], axis=1)
    for li in range(nlayers):
        w = refs[nx + 2 * li][...]
        bb = refs[nx + 2 * li + 1][...]
        h = jnp.dot(h, w, preferred_element_type=_F32) + bb
        if not (last_linear and li == nlayers - 1):
            h = jnp.maximum(h, 0.0)
    out_ref[...] = h


def _mlp_call(xs, layers, last_linear=False):
    rows = xs[0].shape[0]
    cout = layers[-1]["W"].shape[1]
    args = list(xs)
    for l in layers:
        args.append(l["W"])
        args.append(l["b"].reshape(1, -1))
    return pl.pallas_call(
        functools.partial(_mlp_body, nx=len(xs), nlayers=len(layers),
                          last_linear=last_linear),
        out_shape=jax.ShapeDtypeStruct((rows, cout), _F32),
    )(*args)


def _sel_body(f_ref, xyz_ref, w1_ref, b1_ref, w2_ref, b2_ref,
              cls_ref, nxyz_ref, *, keep):
    f = f_ref[0]
    xyz = xyz_ref[0]
    m = f.shape[0]
    h = jnp.maximum(jnp.dot(f, w1_ref[...], preferred_element_type=_F32)
                    + b1_ref[...], 0.0)
    cls = jnp.dot(h, w2_ref[...], preferred_element_type=_F32) + b2_ref[...]
    cls_ref[0] = cls
    s = jnp.max(jax.nn.sigmoid(cls), axis=1, keepdims=True)
    eye = (jax.lax.broadcasted_iota(jnp.int32, (m, m), 0)
           == jax.lax.broadcasted_iota(jnp.int32, (m, m), 1))
    st = jnp.sum(jnp.where(eye, s, 0.0), axis=0, keepdims=True)
    ii = jax.lax.broadcasted_iota(jnp.int32, (m, m), 0)
    jj = jax.lax.broadcasted_iota(jnp.int32, (m, m), 1)
    beats = (s > st) | ((s == st) & (ii < jj))
    rank = jnp.sum(beats.astype(_F32), axis=0, keepdims=True)
    perm = (jax.lax.broadcasted_iota(jnp.int32, (keep, m), 0)
            == rank.astype(jnp.int32))
    nxyz_ref[0] = _exact_gather(perm.astype(_F32), xyz)


def _select_topk(f2, xyz2, conf_layers, keep):
    b, m, c = f2.shape
    (l1, l2) = conf_layers
    cls, nxyz = pl.pallas_call(
        functools.partial(_sel_body, keep=keep),
        grid=(b,),
        in_specs=[
            pl.BlockSpec((1, m, c), lambda i: (i, 0, 0)),
            pl.BlockSpec((1, m, 3), lambda i: (i, 0, 0)),
            pl.BlockSpec(l1["W"].shape, lambda i: (0, 0)),
            pl.BlockSpec((1, l1["W"].shape[1]), lambda i: (0, 0)),
            pl.BlockSpec(l2["W"].shape, lambda i: (0, 0)),
            pl.BlockSpec((1, l2["W"].shape[1]), lambda i: (0, 0)),
        ],
        out_specs=(pl.BlockSpec((1, m, l2["W"].shape[1]),
                                lambda i: (i, 0, 0)),
                   pl.BlockSpec((1, keep, 3), lambda i: (i, 0, 0))),
        out_shape=(jax.ShapeDtypeStruct((b, m, l2["W"].shape[1]), _F32),
                   jax.ShapeDtypeStruct((b, keep, 3), _F32)),
    )(f2, xyz2, l1["W"], l1["b"].reshape(1, -1),
      l2["W"], l2["b"].reshape(1, -1))
    return cls, nxyz


def _head_body(f_ref, xyz_ref, cw1, cb1, cw2, cb2, vw, vb, ow, ob, mt_ref,
               cls_ref, off_ref, ctr_ref):
    f = f_ref[0]
    xyz = xyz_ref[0]
    h = jnp.maximum(jnp.dot(f, cw1[...], preferred_element_type=_F32)
                    + cb1[...], 0.0)
    cls_ref[0] = jnp.dot(h, cw2[...], preferred_element_type=_F32) + cb2[...]
    vf = jnp.maximum(jnp.dot(f, vw[...], preferred_element_type=_F32)
                     + vb[...], 0.0)
    off = jnp.dot(vf, ow[...], preferred_element_type=_F32) + ob[...]
    mt = mt_ref[...]
    off = jnp.minimum(jnp.maximum(off, -mt), mt)
    off_ref[0] = off
    ctr_ref[0] = xyz + off


def _vote_head(f3, xyz3, conf_layers, vote_layer, off_layer):
    b, m, c = f3.shape
    (l1, l2) = conf_layers
    nc = l2["W"].shape[1]
    maxtr = jnp.array([[3.0, 3.0, 2.0]], dtype=_F32)
    full = lambda a: pl.BlockSpec(a.shape, lambda i: (0,) * a.ndim)
    args = [f3, xyz3, l1["W"], l1["b"].reshape(1, -1), l2["W"],
            l2["b"].reshape(1, -1), vote_layer["W"],
            vote_layer["b"].reshape(1, -1), off_layer["W"],
            off_layer["b"].reshape(1, -1), maxtr]
    in_specs = [pl.BlockSpec((1, m, c), lambda i: (i, 0, 0)),
                pl.BlockSpec((1, m, 3), lambda i: (i, 0, 0))] + \
               [full(a) for a in args[2:]]
    cls, off, ctr = pl.pallas_call(
        _head_body,
        grid=(b,),
        in_specs=in_specs,
        out_specs=(pl.BlockSpec((1, m, nc), lambda i: (i, 0, 0)),
                   pl.BlockSpec((1, m, 3), lambda i: (i, 0, 0)),
                   pl.BlockSpec((1, m, 3), lambda i: (i, 0, 0))),
        out_shape=(jax.ShapeDtypeStruct((b, m, nc), _F32),
                   jax.ShapeDtypeStruct((b, m, 3), _F32),
                   jax.ShapeDtypeStruct((b, m, 3), _F32)),
    )(*args)
    return cls, off, ctr


def _sa_layer(p, xyz, feats, new_xyz, radii, nsamples, mb):
    b, m, _ = new_xyz.shape
    xyzt = jnp.transpose(xyz, (0, 2, 1))
    pts = jnp.concatenate([xyz, feats], axis=-1)
    pooled = []
    for s in range(len(radii)):
        ps = _sa_scale(xyzt, pts, new_xyz, p["scale%d" % s], radii[s],
                       nsamples[s], mb)
        pooled.append(ps.reshape(b * m, -1))
    out = _mlp_call(pooled, p["agg"])
    return out.reshape(b, m, -1)


def kernel(points, params):
    pts = points.reshape(_B, _N, 5)
    batch_col = pts[:, :, 0]
    xyz = pts[:, :, 1:4]
    feats = pts[:, :, 4:5]

    xyz1 = _fps(xyz, 1024)
    f1 = _sa_layer(params["sa1"], xyz, feats, xyz1, [0.5, 1.0], [16, 32],
                   mb=64)
    xyz2 = _fps(xyz1, 512)
    f2 = _sa_layer(params["sa2"], xyz1, f1, xyz2, [1.0, 2.0], [16, 32],
                   mb=512)
    cls2, xyz3 = _select_topk(f2, xyz2, params["conf2"], 256)
    f3 = _sa_layer(params["sa3"], xyz2, f2, xyz3, [2.0, 4.0], [16, 32],
                   mb=256)
    cls3, offsets, centers = _vote_head(f3, xyz3, params["conf3"],
                                        params["vote_mlp"][0],
                                        params["vote_off"])
    f4 = _sa_layer(params["sa4"], xyz3, f3, centers, [4.0, 8.0], [16, 32],
                   mb=256)

    ctr_b = batch_col[:, :256].reshape(-1, 1)
    centers_out = jnp.concatenate([ctr_b, centers.reshape(-1, 3)], axis=1)
    centers_origin_out = jnp.concatenate([ctr_b, xyz3.reshape(-1, 3)], axis=1)
    ctr_offsets_out = jnp.concatenate([ctr_b, offsets.reshape(-1, 3)], axis=1)
    centers_features = f4.reshape(-1, f4.shape[-1])
    return (centers_out, centers_origin_out, ctr_offsets_out,
            centers_features, cls2, cls3)


# speedup vs baseline: 4.3151x; 1.1016x over previous
import functools

import jax
import jax.numpy as jnp
from jax.experimental import pallas as pl

_B = 2
_N = 8192
_F32 = jnp.float32
_PREC = None


def _split3(a):
    a1 = a.astype(jnp.bfloat16).astype(_F32)
    r1 = a - a1
    a2 = r1.astype(jnp.bfloat16).astype(_F32)
    a3 = r1 - a2
    return a1, a2, a3


def _exact_gather(onehot, a):
    a1, a2, a3 = _split3(a)
    d = functools.partial(jnp.dot, preferred_element_type=_F32)
    return (d(onehot, a1) + d(onehot, a2)) + d(onehot, a3)


def _fps_body(x_ref, y_ref, z_ref, ox_ref, oy_ref, oz_ref, *, npoint, n):
    x = x_ref[0]
    y = y_ref[0]
    z = z_ref[0]
    rows = x.shape[0]
    lin = (jax.lax.broadcasted_iota(jnp.int32, (rows, 128), 0) * 128
           + jax.lax.broadcasted_iota(jnp.int32, (rows, 128), 1))
    orows = npoint // 128
    olin = (jax.lax.broadcasted_iota(jnp.int32, (orows, 128), 0) * 128
            + jax.lax.broadcasted_iota(jnp.int32, (orows, 128), 1))

    lane = jax.lax.broadcasted_iota(jnp.int32, (1, 128), 1)

    def body(i, state):
        dists, far, ox, oy, oz = state
        row = far // 128
        col = far % 128
        sel = lane == col
        cx = jnp.sum(jnp.where(sel, x_ref[0, pl.ds(row, 1), :], 0.0))
        cy = jnp.sum(jnp.where(sel, y_ref[0, pl.ds(row, 1), :], 0.0))
        cz = jnp.sum(jnp.where(sel, z_ref[0, pl.ds(row, 1), :], 0.0))
        dx = x - cx
        dy = y - cy
        dz = z - cz
        d = dx * dx + dy * dy + dz * dz
        dists = jnp.minimum(dists, d)
        step = olin == i
        ox = jnp.where(step, cx, ox)
        oy = jnp.where(step, cy, oy)
        oz = jnp.where(step, cz, oz)
        mx = jnp.max(dists)
        far = jnp.min(jnp.where(dists == mx, lin, n)).astype(jnp.int32)
        return dists, far, ox, oy, oz

    init = (jnp.full((rows, 128), 1e10, _F32), jnp.int32(0),
            jnp.zeros((orows, 128), _F32), jnp.zeros((orows, 128), _F32),
            jnp.zeros((orows, 128), _F32))
    _, _, ox, oy, oz = jax.lax.fori_loop(0, npoint, body, init)
    ox_ref[0] = ox
    oy_ref[0] = oy
    oz_ref[0] = oz


def _fps(xyz, npoint):
    b, n, _ = xyz.shape
    rows = n // 128
    orows = npoint // 128
    xs = xyz[:, :, 0].reshape(b, rows, 128)
    ys = xyz[:, :, 1].reshape(b, rows, 128)
    zs = xyz[:, :, 2].reshape(b, rows, 128)
    spec_in = pl.BlockSpec((1, rows, 128), lambda i: (i, 0, 0))
    spec_out = pl.BlockSpec((1, orows, 128), lambda i: (i, 0, 0))
    out_sd = jax.ShapeDtypeStruct((b, orows, 128), _F32)
    ox, oy, oz = pl.pallas_call(
        functools.partial(_fps_body, npoint=npoint, n=n),
        grid=(b,),
        in_specs=[spec_in, spec_in, spec_in],
        out_specs=(spec_out, spec_out, spec_out),
        out_shape=(out_sd, out_sd, out_sd),
    )(xs, ys, zs)
    return jnp.stack([ox.reshape(b, npoint), oy.reshape(b, npoint),
                      oz.reshape(b, npoint)], axis=-1)


def _sa_scale_body(xt_ref, p_ref, c_ref, *refs, radius, nsample, nlayers):
    out_ref = refs[-1]
    wrefs = refs[:-1]
    xt = xt_ref[0]
    pts = p_ref[0]
    c = c_ref[0]
    n = pts.shape[0]
    mb = c.shape[0]
    cin = pts.shape[1]

    px = xt[0:1, :]
    py = xt[1:2, :]
    pz = xt[2:3, :]
    cx = c[:, 0:1]
    cy = c[:, 1:2]
    cz = c[:, 2:3]
    ddx = cx - px
    ddy = cy - py
    ddz = cz - pz
    dist = ddx * ddx + ddy * ddy + ddz * ddz
    r2 = radius * radius
    cur = jnp.where(dist <= r2, dist, 1e10)

    iota_n = jax.lax.broadcasted_iota(jnp.int32, (mb, n), 1)
    cpad = jnp.concatenate([c, jnp.zeros((mb, cin - 3), _F32)], axis=1)

    p1, p2, p3 = _split3(pts)

    def extract(cur):
        m = jnp.min(cur, axis=1, keepdims=True)
        idx = jnp.min(jnp.where(cur == m, iota_n, n), axis=1,
                      keepdims=True)
        onehot = (iota_n == idx)
        oh = onehot.astype(_F32)
        d = functools.partial(jnp.dot, preferred_element_type=_F32)
        g = (d(oh, p1) + d(oh, p2)) + d(oh, p3)
        return m, onehot, g

    def mlp(g):
        h = g - cpad
        for li in range(nlayers):
            w = wrefs[2 * li][...]
            bb = wrefs[2 * li + 1][...]
            h = jnp.dot(h, w, preferred_element_type=_F32) + bb
            h = jnp.maximum(h, 0.0)
        return h

    m0, onehot0, g0 = extract(cur)
    cur = jnp.where(onehot0, 1e10, cur)
    pooled0 = mlp(g0)

    def body(_, carry):
        cur, pooled = carry
        m, onehot, g = extract(cur)
        cur = jnp.where(onehot, 1e10, cur)
        gsel = jnp.where(m <= r2, g, g0)
        h = mlp(gsel)
        return cur, jnp.maximum(pooled, h)

    _, pooled = jax.lax.fori_loop(1, nsample, body, (cur, pooled0))
    out_ref[0] = pooled


def _sa_scale(xyzt, pts, centers, layers, radius, nsample, mb):
    b, _, n = xyzt.shape
    m = centers.shape[1]
    cout = layers[-1]["W"].shape[1]
    nlayers = len(layers)
    args = [xyzt, pts, centers]
    in_specs = [
        pl.BlockSpec((1, 3, n), lambda i, j: (i, 0, 0)),
        pl.BlockSpec((1, n, pts.shape[2]), lambda i, j: (i, 0, 0)),
        pl.BlockSpec((1, mb, 3), lambda i, j: (i, j, 0)),
    ]
    for l in layers:
        args.append(l["W"])
        args.append(l["b"].reshape(1, -1))
        in_specs.append(pl.BlockSpec(l["W"].shape, lambda i, j: (0, 0)))
        in_specs.append(pl.BlockSpec((1, l["W"].shape[1]),
                                     lambda i, j: (0, 0)))
    out = pl.pallas_call(
        functools.partial(_sa_scale_body, radius=radius, nsample=nsample,
                          nlayers=nlayers),
        grid=(b, m // mb),
        in_specs=in_specs,
        out_specs=pl.BlockSpec((1, mb, cout), lambda i, j: (i, j, 0)),
        out_shape=jax.ShapeDtypeStruct((b, m, cout), _F32),
    )(*args)
    return out


def _mlp_body(*refs, nx, nlayers, last_linear):
    out_ref = refs[-1]
    if nx == 1:
        h = refs[0][...]
    else:
        h = jnp.concatenate([refs[i][...] for i in range(nx)], axis=1)
    for li in range(nlayers):
        w = refs[nx + 2 * li][...]
        bb = refs[nx + 2 * li + 1][...]
        h = jnp.dot(h, w, preferred_element_type=_F32) + bb
        if not (last_linear and li == nlayers - 1):
            h = jnp.maximum(h, 0.0)
    out_ref[...] = h


def _mlp_call(xs, layers, last_linear=False):
    rows = xs[0].shape[0]
    cout = layers[-1]["W"].shape[1]
    args = list(xs)
    for l in layers:
        args.append(l["W"])
        args.append(l["b"].reshape(1, -1))
    return pl.pallas_call(
        functools.partial(_mlp_body, nx=len(xs), nlayers=len(layers),
                          last_linear=last_linear),
        out_shape=jax.ShapeDtypeStruct((rows, cout), _F32),
    )(*args)


def _sel_body(f_ref, xyz_ref, w1_ref, b1_ref, w2_ref, b2_ref,
              cls_ref, nxyz_ref, *, keep):
    f = f_ref[0]
    xyz = xyz_ref[0]
    m = f.shape[0]
    h = jnp.maximum(jnp.dot(f, w1_ref[...], preferred_element_type=_F32)
                    + b1_ref[...], 0.0)
    cls = jnp.dot(h, w2_ref[...], preferred_element_type=_F32) + b2_ref[...]
    cls_ref[0] = cls
    s = jnp.max(jax.nn.sigmoid(cls), axis=1, keepdims=True)
    eye = (jax.lax.broadcasted_iota(jnp.int32, (m, m), 0)
           == jax.lax.broadcasted_iota(jnp.int32, (m, m), 1))
    st = jnp.sum(jnp.where(eye, s, 0.0), axis=0, keepdims=True)
    ii = jax.lax.broadcasted_iota(jnp.int32, (m, m), 0)
    jj = jax.lax.broadcasted_iota(jnp.int32, (m, m), 1)
    beats = (s > st) | ((s == st) & (ii < jj))
    rank = jnp.sum(beats.astype(_F32), axis=0, keepdims=True)
    perm = (jax.lax.broadcasted_iota(jnp.int32, (keep, m), 0)
            == rank.astype(jnp.int32))
    nxyz_ref[0] = _exact_gather(perm.astype(_F32), xyz)


def _select_topk(f2, xyz2, conf_layers, keep):
    b, m, c = f2.shape
    (l1, l2) = conf_layers
    cls, nxyz = pl.pallas_call(
        functools.partial(_sel_body, keep=keep),
        grid=(b,),
        in_specs=[
            pl.BlockSpec((1, m, c), lambda i: (i, 0, 0)),
            pl.BlockSpec((1, m, 3), lambda i: (i, 0, 0)),
            pl.BlockSpec(l1["W"].shape, lambda i: (0, 0)),
            pl.BlockSpec((1, l1["W"].shape[1]), lambda i: (0, 0)),
            pl.BlockSpec(l2["W"].shape, lambda i: (0, 0)),
            pl.BlockSpec((1, l2["W"].shape[1]), lambda i: (0, 0)),
        ],
        out_specs=(pl.BlockSpec((1, m, l2["W"].shape[1]),
                                lambda i: (i, 0, 0)),
                   pl.BlockSpec((1, keep, 3), lambda i: (i, 0, 0))),
        out_shape=(jax.ShapeDtypeStruct((b, m, l2["W"].shape[1]), _F32),
                   jax.ShapeDtypeStruct((b, keep, 3), _F32)),
    )(f2, xyz2, l1["W"], l1["b"].reshape(1, -1),
      l2["W"], l2["b"].reshape(1, -1))
    return cls, nxyz


def _head_body(f_ref, xyz_ref, cw1, cb1, cw2, cb2, vw, vb, ow, ob, mt_ref,
               cls_ref, off_ref, ctr_ref):
    f = f_ref[0]
    xyz = xyz_ref[0]
    h = jnp.maximum(jnp.dot(f, cw1[...], preferred_element_type=_F32)
                    + cb1[...], 0.0)
    cls_ref[0] = jnp.dot(h, cw2[...], preferred_element_type=_F32) + cb2[...]
    vf = jnp.maximum(jnp.dot(f, vw[...], preferred_element_type=_F32)
                     + vb[...], 0.0)
    off = jnp.dot(vf, ow[...], preferred_element_type=_F32) + ob[...]
    mt = mt_ref[...]
    off = jnp.minimum(jnp.maximum(off, -mt), mt)
    off_ref[0] = off
    ctr_ref[0] = xyz + off


def _vote_head(f3, xyz3, conf_layers, vote_layer, off_layer):
    b, m, c = f3.shape
    (l1, l2) = conf_layers
    nc = l2["W"].shape[1]
    maxtr = jnp.array([[3.0, 3.0, 2.0]], dtype=_F32)
    full = lambda a: pl.BlockSpec(a.shape, lambda i: (0,) * a.ndim)
    args = [f3, xyz3, l1["W"], l1["b"].reshape(1, -1), l2["W"],
            l2["b"].reshape(1, -1), vote_layer["W"],
            vote_layer["b"].reshape(1, -1), off_layer["W"],
            off_layer["b"].reshape(1, -1), maxtr]
    in_specs = [pl.BlockSpec((1, m, c), lambda i: (i, 0, 0)),
                pl.BlockSpec((1, m, 3), lambda i: (i, 0, 0))] + \
               [full(a) for a in args[2:]]
    cls, off, ctr = pl.pallas_call(
        _head_body,
        grid=(b,),
        in_specs=in_specs,
        out_specs=(pl.BlockSpec((1, m, nc), lambda i: (i, 0, 0)),
                   pl.BlockSpec((1, m, 3), lambda i: (i, 0, 0)),
                   pl.BlockSpec((1, m, 3), lambda i: (i, 0, 0))),
        out_shape=(jax.ShapeDtypeStruct((b, m, nc), _F32),
                   jax.ShapeDtypeStruct((b, m, 3), _F32),
                   jax.ShapeDtypeStruct((b, m, 3), _F32)),
    )(*args)
    return cls, off, ctr


def _sa_layer(p, xyz, feats, new_xyz, radii, nsamples, mb):
    b, m, _ = new_xyz.shape
    xyzt = jnp.transpose(xyz, (0, 2, 1))
    pts = jnp.concatenate([xyz, feats], axis=-1)
    pooled = []
    for s in range(len(radii)):
        ps = _sa_scale(xyzt, pts, new_xyz, p["scale%d" % s], radii[s],
                       nsamples[s], mb)
        pooled.append(ps.reshape(b * m, -1))
    out = _mlp_call(pooled, p["agg"])
    return out.reshape(b, m, -1)


def kernel(points, params):
    pts = points.reshape(_B, _N, 5)
    batch_col = pts[:, :, 0]
    xyz = pts[:, :, 1:4]
    feats = pts[:, :, 4:5]

    xyz1 = _fps(xyz, 1024)
    f1 = _sa_layer(params["sa1"], xyz, feats, xyz1, [0.5, 1.0], [16, 32],
                   mb=128)
    xyz2 = _fps(xyz1, 512)
    f2 = _sa_layer(params["sa2"], xyz1, f1, xyz2, [1.0, 2.0], [16, 32],
                   mb=512)
    cls2, xyz3 = _select_topk(f2, xyz2, params["conf2"], 256)
    f3 = _sa_layer(params["sa3"], xyz2, f2, xyz3, [2.0, 4.0], [16, 32],
                   mb=256)
    cls3, offsets, centers = _vote_head(f3, xyz3, params["conf3"],
                                        params["vote_mlp"][0],
                                        params["vote_off"])
    f4 = _sa_layer(params["sa4"], xyz3, f3, centers, [4.0, 8.0], [16, 32],
                   mb=256)

    ctr_b = batch_col[:, :256].reshape(-1, 1)
    centers_out = jnp.concatenate([ctr_b, centers.reshape(-1, 3)], axis=1)
    centers_origin_out = jnp.concatenate([ctr_b, xyz3.reshape(-1, 3)], axis=1)
    ctr_offsets_out = jnp.concatenate([ctr_b, offsets.reshape(-1, 3)], axis=1)
    centers_features = f4.reshape(-1, f4.shape[-1])
    return (centers_out, centers_origin_out, ctr_offsets_out,
            centers_features, cls2, cls3)


# speedup vs baseline: 5.3511x; 1.2401x over previous
import functools

import jax
import jax.numpy as jnp
from jax.experimental import pallas as pl

_B = 2
_N = 8192
_F32 = jnp.float32
_PREC = None


def _split3(a):
    a1 = a.astype(jnp.bfloat16).astype(_F32)
    r1 = a - a1
    a2 = r1.astype(jnp.bfloat16).astype(_F32)
    a3 = r1 - a2
    return a1, a2, a3


def _exact_gather(onehot, a):
    a1, a2, a3 = _split3(a)
    d = functools.partial(jnp.dot, preferred_element_type=_F32)
    return (d(onehot, a1) + d(onehot, a2)) + d(onehot, a3)


def _fps_body(x_ref, y_ref, z_ref, ox_ref, oy_ref, oz_ref, *, npoint, n, nb):
    rows = n // 128
    orows = npoint // 128
    xs = [x_ref[bi] for bi in range(nb)]
    ys = [y_ref[bi] for bi in range(nb)]
    zs = [z_ref[bi] for bi in range(nb)]
    lin = (jax.lax.broadcasted_iota(jnp.int32, (rows, 128), 0) * 128
           + jax.lax.broadcasted_iota(jnp.int32, (rows, 128), 1))
    olin = (jax.lax.broadcasted_iota(jnp.int32, (orows, 128), 0) * 128
            + jax.lax.broadcasted_iota(jnp.int32, (orows, 128), 1))
    lane = jax.lax.broadcasted_iota(jnp.int32, (1, 128), 1)

    def body(i, state):
        dists, fars, outs = state
        step = olin == i
        ndists, nfars, nouts = [], [], []
        for bi in range(nb):
            far = fars[bi]
            row = far // 128
            col = far % 128
            sel = lane == col
            cx = jnp.sum(jnp.where(sel, x_ref[bi, pl.ds(row, 1), :], 0.0))
            cy = jnp.sum(jnp.where(sel, y_ref[bi, pl.ds(row, 1), :], 0.0))
            cz = jnp.sum(jnp.where(sel, z_ref[bi, pl.ds(row, 1), :], 0.0))
            dx = xs[bi] - cx
            dy = ys[bi] - cy
            dz = zs[bi] - cz
            d = dx * dx + dy * dy + dz * dz
            db = jnp.minimum(dists[bi], d)
            mx = jnp.max(db)
            nfar = jnp.min(jnp.where(db == mx, lin, n)).astype(jnp.int32)
            ox, oy, oz = outs[bi]
            ox = jnp.where(step, cx, ox)
            oy = jnp.where(step, cy, oy)
            oz = jnp.where(step, cz, oz)
            ndists.append(db)
            nfars.append(nfar)
            nouts.append([ox, oy, oz])
        return ndists, nfars, nouts

    zo = jnp.zeros((orows, 128), _F32)
    init = ([jnp.full((rows, 128), 1e10, _F32) for _ in range(nb)],
            [jnp.int32(0) for _ in range(nb)],
            [[zo, zo, zo] for _ in range(nb)])
    _, _, outs = jax.lax.fori_loop(0, npoint, body, init)
    for bi in range(nb):
        ox_ref[bi] = outs[bi][0]
        oy_ref[bi] = outs[bi][1]
        oz_ref[bi] = outs[bi][2]


def _fps(xyz, npoint):
    b, n, _ = xyz.shape
    rows = n // 128
    orows = npoint // 128
    xs = xyz[:, :, 0].reshape(b, rows, 128)
    ys = xyz[:, :, 1].reshape(b, rows, 128)
    zs = xyz[:, :, 2].reshape(b, rows, 128)
    out_sd = jax.ShapeDtypeStruct((b, orows, 128), _F32)
    ox, oy, oz = pl.pallas_call(
        functools.partial(_fps_body, npoint=npoint, n=n, nb=b),
        out_shape=(out_sd, out_sd, out_sd),
    )(xs, ys, zs)
    return jnp.stack([ox.reshape(b, npoint), oy.reshape(b, npoint),
                      oz.reshape(b, npoint)], axis=-1)


def _sa_scale_body(xt_ref, p_ref, c_ref, *refs, radius, nsample, nlayers):
    out_ref = refs[-1]
    wrefs = refs[:-1]
    xt = xt_ref[0]
    pts = p_ref[0]
    c = c_ref[0]
    n = pts.shape[0]
    mb = c.shape[0]
    cin = pts.shape[1]

    px = xt[0:1, :]
    py = xt[1:2, :]
    pz = xt[2:3, :]
    cx = c[:, 0:1]
    cy = c[:, 1:2]
    cz = c[:, 2:3]
    ddx = cx - px
    ddy = cy - py
    ddz = cz - pz
    dist = ddx * ddx + ddy * ddy + ddz * ddz
    r2 = radius * radius
    cur = jnp.where(dist <= r2, dist, 1e10)

    iota_n = jax.lax.broadcasted_iota(jnp.int32, (mb, n), 1)
    cpad = jnp.concatenate([c, jnp.zeros((mb, cin - 3), _F32)], axis=1)

    p1, p2, p3 = _split3(pts)
    pcat = jnp.concatenate([p1, p2, p3], axis=1)

    def extract(cur):
        m = jnp.min(cur, axis=1, keepdims=True)
        idx = jnp.min(jnp.where(cur == m, iota_n, n), axis=1,
                      keepdims=True)
        onehot = (iota_n == idx)
        gf = jnp.dot(onehot.astype(_F32), pcat,
                     preferred_element_type=_F32)
        g = (gf[:, :cin] + gf[:, cin:2 * cin]) + gf[:, 2 * cin:]
        return m, onehot, g

    def mlp(g):
        h = g - cpad
        for li in range(nlayers):
            w = wrefs[2 * li][...]
            bb = wrefs[2 * li + 1][...]
            h = jnp.dot(h, w, preferred_element_type=_F32) + bb
            h = jnp.maximum(h, 0.0)
        return h

    m0, onehot0, g0 = extract(cur)
    cur = jnp.where(onehot0, 1e10, cur)
    pooled0 = mlp(g0)

    def body(_, carry):
        cur, pooled = carry
        m, onehot, g = extract(cur)
        cur = jnp.where(onehot, 1e10, cur)
        gsel = jnp.where(m <= r2, g, g0)
        h = mlp(gsel)
        return cur, jnp.maximum(pooled, h)

    _, pooled = jax.lax.fori_loop(1, nsample, body, (cur, pooled0))
    out_ref[0] = pooled


def _sa_scale(xyzt, pts, centers, layers, radius, nsample, mb):
    b, _, n = xyzt.shape
    m = centers.shape[1]
    cout = layers[-1]["W"].shape[1]
    nlayers = len(layers)
    args = [xyzt, pts, centers]
    in_specs = [
        pl.BlockSpec((1, 3, n), lambda i, j: (i, 0, 0)),
        pl.BlockSpec((1, n, pts.shape[2]), lambda i, j: (i, 0, 0)),
        pl.BlockSpec((1, mb, 3), lambda i, j: (i, j, 0)),
    ]
    for l in layers:
        args.append(l["W"])
        args.append(l["b"].reshape(1, -1))
        in_specs.append(pl.BlockSpec(l["W"].shape, lambda i, j: (0, 0)))
        in_specs.append(pl.BlockSpec((1, l["W"].shape[1]),
                                     lambda i, j: (0, 0)))
    out = pl.pallas_call(
        functools.partial(_sa_scale_body, radius=radius, nsample=nsample,
                          nlayers=nlayers),
        grid=(b, m // mb),
        in_specs=in_specs,
        out_specs=pl.BlockSpec((1, mb, cout), lambda i, j: (i, j, 0)),
        out_shape=jax.ShapeDtypeStruct((b, m, cout), _F32),
    )(*args)
    return out


def _mlp_body(*refs, nx, nlayers, last_linear):
    out_ref = refs[-1]
    if nx == 1:
        h = refs[0][...]
    else:
        h = jnp.concatenate([refs[i][...] for i in range(nx)], axis=1)
    for li in range(nlayers):
        w = refs[nx + 2 * li][...]
        bb = refs[nx + 2 * li + 1][...]
        h = jnp.dot(h, w, preferred_element_type=_F32) + bb
        if not (last_linear and li == nlayers - 1):
            h = jnp.maximum(h, 0.0)
    out_ref[...] = h


def _mlp_call(xs, layers, last_linear=False):
    rows = xs[0].shape[0]
    cout = layers[-1]["W"].shape[1]
    args = list(xs)
    for l in layers:
        args.append(l["W"])
        args.append(l["b"].reshape(1, -1))
    return pl.pallas_call(
        functools.partial(_mlp_body, nx=len(xs), nlayers=len(layers),
                          last_linear=last_linear),
        out_shape=jax.ShapeDtypeStruct((rows, cout), _F32),
    )(*args)


def _sel_body(f_ref, xyz_ref, w1_ref, b1_ref, w2_ref, b2_ref,
              cls_ref, nxyz_ref, *, keep):
    f = f_ref[0]
    xyz = xyz_ref[0]
    m = f.shape[0]
    h = jnp.maximum(jnp.dot(f, w1_ref[...], preferred_element_type=_F32)
                    + b1_ref[...], 0.0)
    cls = jnp.dot(h, w2_ref[...], preferred_element_type=_F32) + b2_ref[...]
    cls_ref[0] = cls
    s = jnp.max(jax.nn.sigmoid(cls), axis=1, keepdims=True)
    eye = (jax.lax.broadcasted_iota(jnp.int32, (m, m), 0)
           == jax.lax.broadcasted_iota(jnp.int32, (m, m), 1))
    st = jnp.sum(jnp.where(eye, s, 0.0), axis=0, keepdims=True)
    ii = jax.lax.broadcasted_iota(jnp.int32, (m, m), 0)
    jj = jax.lax.broadcasted_iota(jnp.int32, (m, m), 1)
    beats = (s > st) | ((s == st) & (ii < jj))
    rank = jnp.sum(beats.astype(_F32), axis=0, keepdims=True)
    perm = (jax.lax.broadcasted_iota(jnp.int32, (keep, m), 0)
            == rank.astype(jnp.int32))
    nxyz_ref[0] = _exact_gather(perm.astype(_F32), xyz)


def _select_topk(f2, xyz2, conf_layers, keep):
    b, m, c = f2.shape
    (l1, l2) = conf_layers
    cls, nxyz = pl.pallas_call(
        functools.partial(_sel_body, keep=keep),
        grid=(b,),
        in_specs=[
            pl.BlockSpec((1, m, c), lambda i: (i, 0, 0)),
            pl.BlockSpec((1, m, 3), lambda i: (i, 0, 0)),
            pl.BlockSpec(l1["W"].shape, lambda i: (0, 0)),
            pl.BlockSpec((1, l1["W"].shape[1]), lambda i: (0, 0)),
            pl.BlockSpec(l2["W"].shape, lambda i: (0, 0)),
            pl.BlockSpec((1, l2["W"].shape[1]), lambda i: (0, 0)),
        ],
        out_specs=(pl.BlockSpec((1, m, l2["W"].shape[1]),
                                lambda i: (i, 0, 0)),
                   pl.BlockSpec((1, keep, 3), lambda i: (i, 0, 0))),
        out_shape=(jax.ShapeDtypeStruct((b, m, l2["W"].shape[1]), _F32),
                   jax.ShapeDtypeStruct((b, keep, 3), _F32)),
    )(f2, xyz2, l1["W"], l1["b"].reshape(1, -1),
      l2["W"], l2["b"].reshape(1, -1))
    return cls, nxyz


def _head_body(f_ref, xyz_ref, cw1, cb1, cw2, cb2, vw, vb, ow, ob, mt_ref,
               cls_ref, off_ref, ctr_ref):
    f = f_ref[0]
    xyz = xyz_ref[0]
    h = jnp.maximum(jnp.dot(f, cw1[...], preferred_element_type=_F32)
                    + cb1[...], 0.0)
    cls_ref[0] = jnp.dot(h, cw2[...], preferred_element_type=_F32) + cb2[...]
    vf = jnp.maximum(jnp.dot(f, vw[...], preferred_element_type=_F32)
                     + vb[...], 0.0)
    off = jnp.dot(vf, ow[...], preferred_element_type=_F32) + ob[...]
    mt = mt_ref[...]
    off = jnp.minimum(jnp.maximum(off, -mt), mt)
    off_ref[0] = off
    ctr_ref[0] = xyz + off


def _vote_head(f3, xyz3, conf_layers, vote_layer, off_layer):
    b, m, c = f3.shape
    (l1, l2) = conf_layers
    nc = l2["W"].shape[1]
    maxtr = jnp.array([[3.0, 3.0, 2.0]], dtype=_F32)
    full = lambda a: pl.BlockSpec(a.shape, lambda i: (0,) * a.ndim)
    args = [f3, xyz3, l1["W"], l1["b"].reshape(1, -1), l2["W"],
            l2["b"].reshape(1, -1), vote_layer["W"],
            vote_layer["b"].reshape(1, -1), off_layer["W"],
            off_layer["b"].reshape(1, -1), maxtr]
    in_specs = [pl.BlockSpec((1, m, c), lambda i: (i, 0, 0)),
                pl.BlockSpec((1, m, 3), lambda i: (i, 0, 0))] + \
               [full(a) for a in args[2:]]
    cls, off, ctr = pl.pallas_call(
        _head_body,
        grid=(b,),
        in_specs=in_specs,
        out_specs=(pl.BlockSpec((1, m, nc), lambda i: (i, 0, 0)),
                   pl.BlockSpec((1, m, 3), lambda i: (i, 0, 0)),
                   pl.BlockSpec((1, m, 3), lambda i: (i, 0, 0))),
        out_shape=(jax.ShapeDtypeStruct((b, m, nc), _F32),
                   jax.ShapeDtypeStruct((b, m, 3), _F32),
                   jax.ShapeDtypeStruct((b, m, 3), _F32)),
    )(*args)
    return cls, off, ctr


def _sa_layer(p, xyz, feats, new_xyz, radii, nsamples, mb):
    b, m, _ = new_xyz.shape
    xyzt = jnp.transpose(xyz, (0, 2, 1))
    pts = jnp.concatenate([xyz, feats], axis=-1)
    pooled = []
    for s in range(len(radii)):
        ps = _sa_scale(xyzt, pts, new_xyz, p["scale%d" % s], radii[s],
                       nsamples[s], mb)
        pooled.append(ps.reshape(b * m, -1))
    out = _mlp_call(pooled, p["agg"])
    return out.reshape(b, m, -1)


def kernel(points, params):
    pts = points.reshape(_B, _N, 5)
    batch_col = pts[:, :, 0]
    xyz = pts[:, :, 1:4]
    feats = pts[:, :, 4:5]

    xyz1 = _fps(xyz, 1024)
    f1 = _sa_layer(params["sa1"], xyz, feats, xyz1, [0.5, 1.0], [16, 32],
                   mb=128)
    xyz2 = _fps(xyz1, 512)
    f2 = _sa_layer(params["sa2"], xyz1, f1, xyz2, [1.0, 2.0], [16, 32],
                   mb=512)
    cls2, xyz3 = _select_topk(f2, xyz2, params["conf2"], 256)
    f3 = _sa_layer(params["sa3"], xyz2, f2, xyz3, [2.0, 4.0], [16, 32],
                   mb=256)
    cls3, offsets, centers = _vote_head(f3, xyz3, params["conf3"],
                                        params["vote_mlp"][0],
                                        params["vote_off"])
    f4 = _sa_layer(params["sa4"], xyz3, f3, centers, [4.0, 8.0], [16, 32],
                   mb=256)

    ctr_b = batch_col[:, :256].reshape(-1, 1)
    centers_out = jnp.concatenate([ctr_b, centers.reshape(-1, 3)], axis=1)
    centers_origin_out = jnp.concatenate([ctr_b, xyz3.reshape(-1, 3)], axis=1)
    ctr_offsets_out = jnp.concatenate([ctr_b, offsets.reshape(-1, 3)], axis=1)
    centers_features = f4.reshape(-1, f4.shape[-1])
    return (centers_out, centers_origin_out, ctr_offsets_out,
            centers_features, cls2, cls3)
